# baseline (speedup 1.0000x reference)
import functools

import jax
import jax.numpy as jnp
from jax import lax
from jax.experimental import pallas as pl
from jax.experimental.pallas import tpu as pltpu
from jax.experimental.pallas import tpu_sc as plsc

N = 10000
E = 640000
D = 128

NC = 2
NS = 16
NW = NC * NS

CH = 128
GCH = 16
TOTCH = 5120
CPT = TOTCH // NW
NGRP = CPT // GCH
EPAD = TOTCH * CH
NP = 10240
RPS = NP // NS
BN = 1280


def _sc_aggregate(table, srcs, dsts, with_counts):
  out_type = [jax.ShapeDtypeStruct((NC, NP, D), jnp.float32)]
  scratch = [
      pltpu.VMEM((GCH, CH), jnp.int32),
      pltpu.VMEM((GCH, CH), jnp.int32),
      pltpu.VMEM((2, CH, D), jnp.float32),
      pltpu.VMEM_SHARED((NP, D), jnp.float32),
      pltpu.SemaphoreType.DMA,
      pltpu.SemaphoreType.DMA,
      pltpu.SemaphoreType.DMA,
      pltpu.SemaphoreType.DMA,
  ]
  if with_counts:
    out_type.append(jax.ShapeDtypeStruct((NW, NP), jnp.float32))
    scratch.append(pltpu.VMEM((NP,), jnp.float32))

  mesh = plsc.VectorSubcoreMesh(core_axis_name="c", subcore_axis_name="s")

  def body(table_h, srcs_h, dsts_h, *rest):
    if with_counts:
      p_h, c_h, src_v, dst_v, rows_v, acc_sh, sga, sgb, ssa, ssb, cnt_v = rest
    else:
      p_h, src_v, dst_v, rows_v, acc_sh, sga, sgb, ssa, ssb = rest
    semg = (sga, sgb)
    sems = (ssa, ssb)
    cid = lax.axis_index("c")
    sid = lax.axis_index("s")
    wid = sid * NC + cid
    start_chunk = wid * CPT

    zeros16 = jnp.zeros((16,), jnp.float32)
    @pl.loop(0, CH)
    def _(i):
      for k in range(D // 16):
        rows_v[0, i, pl.ds(k * 16, 16)] = zeros16
    if with_counts:
      @pl.loop(0, NP // 16)
      def _(i):
        cnt_v[pl.ds(i * 16, 16)] = zeros16

    for b in range(RPS // CH):
      r0 = sid * RPS + b * CH
      pltpu.sync_copy(rows_v.at[0], acc_sh.at[pl.ds(r0, CH)])

    plsc.subcore_barrier()

    ones16 = jnp.ones((16,), jnp.float32)

    @pl.loop(0, NGRP)
    def _(g):
      c0 = start_chunk + g * GCH
      pltpu.sync_copy(srcs_h.at[pl.ds(c0, GCH)], src_v)
      pltpu.sync_copy(dsts_h.at[pl.ds(c0, GCH)], dst_v)

      cps = [None] * GCH
      ss = [None] * GCH
      cps[0] = pltpu.async_copy(table_h.at[src_v.at[0]], rows_v.at[0],
                                semg[0])
      if with_counts:
        for j in range(GCH):
          for k in range(CH // 16):
            idx = dst_v[j, pl.ds(k * 16, 16)]
            plsc.addupdate_scatter(cnt_v, [idx], ones16)
      for j in range(GCH):
        b = j & 1
        cps[j].wait()
        ss[j] = pltpu.async_copy(rows_v.at[b], acc_sh.at[dst_v.at[j]],
                                 sems[b], add=True)
        if j + 1 < GCH:
          if j >= 1:
            ss[j - 1].wait()
          cps[j + 1] = pltpu.async_copy(table_h.at[src_v.at[j + 1]],
                                        rows_v.at[1 - b], semg[1 - b])
      ss[GCH - 2].wait()
      ss[GCH - 1].wait()

    plsc.subcore_barrier()

    r0 = sid * RPS
    pltpu.sync_copy(acc_sh.at[pl.ds(r0, RPS)], p_h.at[cid, pl.ds(r0, RPS)])
    if with_counts:
      pltpu.sync_copy(cnt_v, c_h.at[wid])

  k = pl.kernel(body, out_type=tuple(out_type), mesh=mesh,
                scratch_types=tuple(scratch),
                compiler_params=pltpu.CompilerParams(
                    needs_layout_passes=False))
  return k(table, srcs, dsts)


def _tc_layer_body(relu, p_ref, c_ref, x_ref, wl_ref, wr_ref, b_ref, o_ref):
  s = p_ref[0] + p_ref[1]
  cnt = jnp.sum(c_ref[...], axis=0)[:, None]
  mean = s / jnp.maximum(cnt, 1.0)
  acc = (jnp.dot(mean, wl_ref[...], preferred_element_type=jnp.float32)
         + jnp.dot(x_ref[...], wr_ref[...], preferred_element_type=jnp.float32)
         + b_ref[...])
  o_ref[...] = jnp.maximum(acc, 0.0) if relu else acc


def _tc_layer(p, c, x, wlt, wrt, b, relu):
  return pl.pallas_call(
      functools.partial(_tc_layer_body, relu),
      grid=(NP // BN,),
      in_specs=[
          pl.BlockSpec((NC, BN, D), lambda i: (0, i, 0)),
          pl.BlockSpec((NW, BN), lambda i: (0, i)),
          pl.BlockSpec((BN, D), lambda i: (i, 0)),
          pl.BlockSpec((D, D), lambda i: (0, 0)),
          pl.BlockSpec((D, D), lambda i: (0, 0)),
          pl.BlockSpec((1, D), lambda i: (0, 0)),
      ],
      out_specs=pl.BlockSpec((BN, D), lambda i: (i, 0)),
      out_shape=jax.ShapeDtypeStruct((NP, D), jnp.float32),
  )(p, c, x, wlt, wrt, b)


def kernel(edge_index, emb, W1l, b1l, W1r, W2l, b2l, W2r):
  src = edge_index[0]
  dst = edge_index[1]
  pad = EPAD - E
  srcs = jnp.concatenate(
      [src, jnp.zeros((pad,), jnp.int32)]).reshape(TOTCH, CH)
  pad_dst = N + jnp.arange(pad, dtype=jnp.int32) % (NP - N)
  dsts = jnp.concatenate([dst, pad_dst]).reshape(TOTCH, CH)
  embp = jnp.pad(emb, ((0, NP - N), (0, 0)))

  p1, c = _sc_aggregate(embp, srcs, dsts, with_counts=True)
  h = _tc_layer(p1, c, embp, W1l.T, W1r.T, b1l.reshape(1, D), relu=True)
  (p2,) = _sc_aggregate(h, srcs, dsts, with_counts=False)
  out = _tc_layer(p2, c, h, W2l.T, W2r.T, b2l.reshape(1, D), relu=False)
  return out[:N]


# speedup vs baseline: 1.8516x; 1.8516x over previous
import functools

import jax
import jax.numpy as jnp
from jax import lax
from jax.experimental import pallas as pl
from jax.experimental.pallas import tpu as pltpu
from jax.experimental.pallas import tpu_sc as plsc

N = 10000
E = 640000
D = 128

NC = 2
NS = 16
NW = NC * NS

CH = 128
GCH = 16
TOTCH = 5120
EPAD = TOTCH * CH
NP = 10240
HALF = NP // 2
AROWS = HALF + CH
CPS = TOTCH // NS
NGRP = CPS // GCH
EBUF = GCH * CH + 16
CPT = TOTCH // NW
BN = 1280


def _sc_counts(dsts):
  mesh = plsc.VectorSubcoreMesh(core_axis_name="c", subcore_axis_name="s")

  def body(dsts_h, c_h, dst_v, cnt_v):
    cid = lax.axis_index("c")
    sid = lax.axis_index("s")
    wid = sid * NC + cid
    start_chunk = wid * CPT

    zeros16 = jnp.zeros((16,), jnp.float32)
    @pl.loop(0, NP // 16)
    def _(i):
      cnt_v[pl.ds(i * 16, 16)] = zeros16

    ones16 = jnp.ones((16,), jnp.float32)
    @pl.loop(0, CPT // GCH)
    def _(g):
      pltpu.sync_copy(dsts_h.at[pl.ds(start_chunk + g * GCH, GCH)], dst_v)
      for j in range(GCH):
        for k in range(CH // 16):
          idx = dst_v[j, pl.ds(k * 16, 16)]
          plsc.addupdate_scatter(cnt_v, [idx], ones16)

    pltpu.sync_copy(cnt_v, c_h.at[wid])

  k = pl.kernel(body,
                out_type=jax.ShapeDtypeStruct((NW, NP), jnp.float32),
                mesh=mesh,
                scratch_types=(
                    pltpu.VMEM((GCH, CH), jnp.int32),
                    pltpu.VMEM((NP,), jnp.float32),
                ),
                compiler_params=pltpu.CompilerParams(
                    needs_layout_passes=False))
  return k(dsts)


def _sc_aggregate(table, srcs, dsts):
  mesh = plsc.VectorSubcoreMesh(core_axis_name="c", subcore_axis_name="s")

  def body(table_h, srcs_h, dsts_h, p_h, src_v, dst_v, rows_v, srow_v,
           esrc_v, edst_v, table_sh, acc_sh, semg, sems):
    cid = lax.axis_index("c")
    sid = lax.axis_index("s")
    c_lo = cid * HALF

    pltpu.sync_copy(table_h.at[pl.ds(c_lo + sid * (HALF // NS), HALF // NS)],
                    table_sh.at[pl.ds(sid * (HALF // NS), HALF // NS)])

    def drain(sem):
      pltpu.make_async_copy(table_h.at[pl.ds(0, CH)], rows_v.at[0],
                            sem).wait()

    trash16 = HALF + jnp.arange(16, dtype=jnp.int32)
    zeros16 = jnp.zeros((16,), jnp.float32)
    zeros16i = jnp.zeros((16,), jnp.int32)

    for q in range(2):
      q_lo = q * HALF

      @pl.loop(0, CH)
      def _(i):
        for k in range(D // 16):
          rows_v[0, i, pl.ds(k * 16, 16)] = zeros16
      for t in range(3):
        zc = sid + NS * t
        @pl.when(zc < AROWS // CH)
        def _():
          pltpu.sync_copy(rows_v.at[0], acc_sh.at[pl.ds(zc * CH, CH)])

      plsc.subcore_barrier()

      @pl.loop(0, NGRP)
      def _(g):
        c0 = sid * CPS + g * GCH
        pltpu.sync_copy(srcs_h.at[pl.ds(c0, GCH)], src_v)
        pltpu.sync_copy(dsts_h.at[pl.ds(c0, GCH)], dst_v)

        cur = jnp.int32(0)
        for j in range(GCH):
          for k in range(CH // 16):
            sv = src_v[j, pl.ds(k * 16, 16)]
            dv = dst_v[j, pl.ds(k * 16, 16)]
            sl = sv - c_lo
            dl = dv - q_lo
            m = ((sl >= 0) & (sl < HALF)) & ((dl >= 0) & (dl < HALF))
            plsc.store_compressed(esrc_v.at[pl.ds(cur, 16)], sl, mask=m)
            plsc.store_compressed(edst_v.at[pl.ds(cur, 16)], dl, mask=m)
            cur = cur + jnp.sum(m.astype(jnp.int32))

        cur_r = ((cur + (CH - 1)) // CH) * CH
        @pl.loop(cur, cur_r, step=16)
        def _(i):
          esrc_v[pl.ds(i, 16)] = zeros16i
          edst_v[pl.ds(i, 16)] = trash16
        nb = cur_r // CH

        @pl.when(nb > 0)
        def _():
          pltpu.async_copy(table_sh.at[esrc_v.at[pl.ds(0, CH)]],
                           rows_v.at[0], semg)

        @pl.loop(0, nb)
        def _(b):
          bb = lax.rem(b, 2)
          drain(semg)
          for k in range(CH // 16):
            srow_v[bb, pl.ds(k * 16, 16)] = edst_v[pl.ds(b * CH + k * 16,
                                                         16)]
          pltpu.async_copy(rows_v.at[bb], acc_sh.at[srow_v.at[bb]], sems,
                           add=True)
          @pl.when(b + 1 < nb)
          def _():
            @pl.when(b >= 1)
            def _():
              drain(sems)
            pltpu.async_copy(table_sh.at[esrc_v.at[pl.ds((b + 1) * CH,
                                                         CH)]],
                             rows_v.at[lax.rem(b + 1, 2)], semg)

        @pl.when(nb >= 2)
        def _():
          drain(sems)
        @pl.when(nb >= 1)
        def _():
          drain(sems)

      plsc.subcore_barrier()

      rps = HALF // NS
      pltpu.sync_copy(acc_sh.at[pl.ds(sid * rps, rps)],
                      p_h.at[cid, pl.ds(q_lo + sid * rps, rps)])
      plsc.subcore_barrier()

  k = pl.kernel(body,
                out_type=jax.ShapeDtypeStruct((NC, NP, D), jnp.float32),
                mesh=mesh,
                scratch_types=(
                    pltpu.VMEM((GCH, CH), jnp.int32),
                    pltpu.VMEM((GCH, CH), jnp.int32),
                    pltpu.VMEM((2, CH, D), jnp.float32),
                    pltpu.VMEM((2, CH), jnp.int32),
                    pltpu.VMEM((EBUF,), jnp.int32),
                    pltpu.VMEM((EBUF,), jnp.int32),
                    pltpu.VMEM_SHARED((HALF, D), jnp.float32),
                    pltpu.VMEM_SHARED((AROWS, D), jnp.float32),
                    pltpu.SemaphoreType.DMA,
                    pltpu.SemaphoreType.DMA,
                ),
                compiler_params=pltpu.CompilerParams(
                    needs_layout_passes=False))
  return k(table, srcs, dsts)


def _tc_layer_body(relu, p_ref, c_ref, x_ref, wl_ref, wr_ref, b_ref, o_ref):
  s = p_ref[0] + p_ref[1]
  cnt = jnp.sum(c_ref[...], axis=0)[:, None]
  mean = s / jnp.maximum(cnt, 1.0)
  acc = (jnp.dot(mean, wl_ref[...], preferred_element_type=jnp.float32)
         + jnp.dot(x_ref[...], wr_ref[...], preferred_element_type=jnp.float32)
         + b_ref[...])
  o_ref[...] = jnp.maximum(acc, 0.0) if relu else acc


def _tc_layer(p, c, x, wlt, wrt, b, relu):
  return pl.pallas_call(
      functools.partial(_tc_layer_body, relu),
      grid=(NP // BN,),
      in_specs=[
          pl.BlockSpec((NC, BN, D), lambda i: (0, i, 0)),
          pl.BlockSpec((NW, BN), lambda i: (0, i)),
          pl.BlockSpec((BN, D), lambda i: (i, 0)),
          pl.BlockSpec((D, D), lambda i: (0, 0)),
          pl.BlockSpec((D, D), lambda i: (0, 0)),
          pl.BlockSpec((1, D), lambda i: (0, 0)),
      ],
      out_specs=pl.BlockSpec((BN, D), lambda i: (i, 0)),
      out_shape=jax.ShapeDtypeStruct((NP, D), jnp.float32),
  )(p, c, x, wlt, wrt, b)


def kernel(edge_index, emb, W1l, b1l, W1r, W2l, b2l, W2r):
  src = edge_index[0]
  dst = edge_index[1]
  pad = EPAD - E
  srcs = jnp.concatenate(
      [src, jnp.zeros((pad,), jnp.int32)]).reshape(TOTCH, CH)
  pad_dst = N + jnp.arange(pad, dtype=jnp.int32) % (NP - N)
  dsts = jnp.concatenate([dst, pad_dst]).reshape(TOTCH, CH)
  embp = jnp.pad(emb, ((0, NP - N), (0, 0)))

  c = _sc_counts(dsts)
  p1 = _sc_aggregate(embp, srcs, dsts)
  h = _tc_layer(p1, c, embp, W1l.T, W1r.T, b1l.reshape(1, D), relu=True)
  p2 = _sc_aggregate(h, srcs, dsts)
  out = _tc_layer(p2, c, h, W2l.T, W2r.T, b2l.reshape(1, D), relu=False)
  return out[:N]


# speedup vs baseline: 2.0674x; 1.1165x over previous
import functools

import jax
import jax.numpy as jnp
from jax import lax
from jax.experimental import pallas as pl
from jax.experimental.pallas import tpu as pltpu
from jax.experimental.pallas import tpu_sc as plsc

N = 10000
E = 640000
D = 128

NC = 2
NS = 16
NW = NC * NS

CH = 128
GCH = 16
TOTCH = 5120
EPAD = TOTCH * CH
NP = 10240
HALF = NP // 2
AROWS = HALF + CH
CPS = TOTCH // NS
NGRP = CPS // GCH
EBUF = GCH * CH + CH + 16
CPT = TOTCH // NW
BN = 1280


def _sc_counts(dsts):
  mesh = plsc.VectorSubcoreMesh(core_axis_name="c", subcore_axis_name="s")

  def body(dsts_h, c_h, dst_v, cnt_v):
    cid = lax.axis_index("c")
    sid = lax.axis_index("s")
    wid = sid * NC + cid
    start_chunk = wid * CPT

    zeros16 = jnp.zeros((16,), jnp.float32)
    @pl.loop(0, NP // 16)
    def _(i):
      cnt_v[pl.ds(i * 16, 16)] = zeros16

    ones16 = jnp.ones((16,), jnp.float32)
    @pl.loop(0, CPT // GCH)
    def _(g):
      pltpu.sync_copy(dsts_h.at[pl.ds(start_chunk + g * GCH, GCH)], dst_v)
      for j in range(GCH):
        for k in range(CH // 16):
          idx = dst_v[j, pl.ds(k * 16, 16)]
          plsc.addupdate_scatter(cnt_v, [idx], ones16)

    pltpu.sync_copy(cnt_v, c_h.at[wid])

  k = pl.kernel(body,
                out_type=jax.ShapeDtypeStruct((NW, NP), jnp.float32),
                mesh=mesh,
                scratch_types=(
                    pltpu.VMEM((GCH, CH), jnp.int32),
                    pltpu.VMEM((NP,), jnp.float32),
                ),
                compiler_params=pltpu.CompilerParams(
                    needs_layout_passes=False))
  return k(dsts)


def _sc_aggregate(table, srcs, dsts):
  mesh = plsc.VectorSubcoreMesh(core_axis_name="c", subcore_axis_name="s")

  def body(table_h, srcs_h, dsts_h, p_h, src_v, dst_v, rows_v, srow_v,
           esrc_v, edst_v, table_sh, acc_sh, semg, sems):
    cid = lax.axis_index("c")
    sid = lax.axis_index("s")
    c_lo = cid * HALF

    pltpu.sync_copy(table_h.at[pl.ds(c_lo + sid * (HALF // NS), HALF // NS)],
                    table_sh.at[pl.ds(sid * (HALF // NS), HALF // NS)])

    def drain(sem):
      pltpu.make_async_copy(table_h.at[pl.ds(0, CH)], rows_v.at[0],
                            sem).wait()

    trash16 = HALF + jnp.arange(16, dtype=jnp.int32)
    zeros16 = jnp.zeros((16,), jnp.float32)
    zeros16i = jnp.zeros((16,), jnp.int32)

    for q in range(2):
      q_lo = q * HALF

      @pl.loop(0, CH)
      def _(i):
        for k in range(D // 16):
          rows_v[0, i, pl.ds(k * 16, 16)] = zeros16
      for t in range(3):
        zc = sid + NS * t
        @pl.when(zc < AROWS // CH)
        def _():
          pltpu.sync_copy(rows_v.at[0], acc_sh.at[pl.ds(zc * CH, CH)])

      plsc.subcore_barrier()

      @pl.loop(0, NGRP, init_carry=jnp.int32(0))
      def rem_out(g, cur):
        c0 = sid * CPS + g * GCH
        pltpu.sync_copy(srcs_h.at[pl.ds(c0, GCH)], src_v)
        pltpu.sync_copy(dsts_h.at[pl.ds(c0, GCH)], dst_v)

        for j in range(GCH):
          for k in range(CH // 16):
            sv = src_v[j, pl.ds(k * 16, 16)]
            dv = dst_v[j, pl.ds(k * 16, 16)]
            sl = sv - c_lo
            dl = dv - q_lo
            m = ((sl >= 0) & (sl < HALF)) & ((dl >= 0) & (dl < HALF))
            plsc.store_compressed(esrc_v.at[pl.ds(cur, 16)], sl, mask=m)
            plsc.store_compressed(edst_v.at[pl.ds(cur, 16)], dl, mask=m)
            cur = cur + jnp.sum(m.astype(jnp.int32))

        nb = cur // CH

        @pl.when(nb > 0)
        def _():
          pltpu.async_copy(table_sh.at[esrc_v.at[pl.ds(0, CH)]],
                           rows_v.at[0], semg)

        @pl.loop(0, nb)
        def _(b):
          bb = lax.rem(b, 2)
          drain(semg)
          for k in range(CH // 16):
            srow_v[bb, pl.ds(k * 16, 16)] = edst_v[pl.ds(b * CH + k * 16,
                                                         16)]
          pltpu.async_copy(rows_v.at[bb], acc_sh.at[srow_v.at[bb]], sems,
                           add=True)
          @pl.when(b + 1 < nb)
          def _():
            @pl.when(b >= 1)
            def _():
              drain(sems)
            pltpu.async_copy(table_sh.at[esrc_v.at[pl.ds((b + 1) * CH,
                                                         CH)]],
                             rows_v.at[lax.rem(b + 1, 2)], semg)

        @pl.when(nb >= 2)
        def _():
          drain(sems)
        @pl.when(nb >= 1)
        def _():
          drain(sems)

        rem = cur - nb * CH
        @pl.when(nb > 0)
        def _():
          @pl.loop(0, (rem + 15) // 16)
          def _(i):
            esrc_v[pl.ds(i * 16, 16)] = esrc_v[pl.ds(nb * CH + i * 16, 16)]
            edst_v[pl.ds(i * 16, 16)] = edst_v[pl.ds(nb * CH + i * 16, 16)]
        return rem

      @pl.when(rem_out > 0)
      def _():
        @pl.loop(rem_out, CH, step=16)
        def _(i):
          esrc_v[pl.ds(i, 16)] = zeros16i
          edst_v[pl.ds(i, 16)] = trash16
        pltpu.sync_copy(table_sh.at[esrc_v.at[pl.ds(0, CH)]], rows_v.at[0])
        for k in range(CH // 16):
          srow_v[0, pl.ds(k * 16, 16)] = edst_v[pl.ds(k * 16, 16)]
        pltpu.sync_copy(rows_v.at[0], acc_sh.at[srow_v.at[0]], add=True)

      plsc.subcore_barrier()

      rps = HALF // NS
      pltpu.sync_copy(acc_sh.at[pl.ds(sid * rps, rps)],
                      p_h.at[cid, pl.ds(q_lo + sid * rps, rps)])
      plsc.subcore_barrier()

  k = pl.kernel(body,
                out_type=jax.ShapeDtypeStruct((NC, NP, D), jnp.float32),
                mesh=mesh,
                scratch_types=(
                    pltpu.VMEM((GCH, CH), jnp.int32),
                    pltpu.VMEM((GCH, CH), jnp.int32),
                    pltpu.VMEM((2, CH, D), jnp.float32),
                    pltpu.VMEM((2, CH), jnp.int32),
                    pltpu.VMEM((EBUF,), jnp.int32),
                    pltpu.VMEM((EBUF,), jnp.int32),
                    pltpu.VMEM_SHARED((HALF, D), jnp.float32),
                    pltpu.VMEM_SHARED((AROWS, D), jnp.float32),
                    pltpu.SemaphoreType.DMA,
                    pltpu.SemaphoreType.DMA,
                ),
                compiler_params=pltpu.CompilerParams(
                    needs_layout_passes=False))
  return k(table, srcs, dsts)


def _tc_layer_body(relu, p_ref, c_ref, x_ref, wl_ref, wr_ref, b_ref, o_ref):
  s = p_ref[0] + p_ref[1]
  cnt = jnp.sum(c_ref[...], axis=0)[:, None]
  mean = s / jnp.maximum(cnt, 1.0)
  acc = (jnp.dot(mean, wl_ref[...], preferred_element_type=jnp.float32)
         + jnp.dot(x_ref[...], wr_ref[...], preferred_element_type=jnp.float32)
         + b_ref[...])
  o_ref[...] = jnp.maximum(acc, 0.0) if relu else acc


def _tc_layer(p, c, x, wlt, wrt, b, relu):
  return pl.pallas_call(
      functools.partial(_tc_layer_body, relu),
      grid=(NP // BN,),
      in_specs=[
          pl.BlockSpec((NC, BN, D), lambda i: (0, i, 0)),
          pl.BlockSpec((NW, BN), lambda i: (0, i)),
          pl.BlockSpec((BN, D), lambda i: (i, 0)),
          pl.BlockSpec((D, D), lambda i: (0, 0)),
          pl.BlockSpec((D, D), lambda i: (0, 0)),
          pl.BlockSpec((1, D), lambda i: (0, 0)),
      ],
      out_specs=pl.BlockSpec((BN, D), lambda i: (i, 0)),
      out_shape=jax.ShapeDtypeStruct((NP, D), jnp.float32),
  )(p, c, x, wlt, wrt, b)


def kernel(edge_index, emb, W1l, b1l, W1r, W2l, b2l, W2r):
  src = edge_index[0]
  dst = edge_index[1]
  pad = EPAD - E
  srcs = jnp.concatenate(
      [src, jnp.zeros((pad,), jnp.int32)]).reshape(TOTCH, CH)
  pad_dst = N + jnp.arange(pad, dtype=jnp.int32) % (NP - N)
  dsts = jnp.concatenate([dst, pad_dst]).reshape(TOTCH, CH)
  embp = jnp.pad(emb, ((0, NP - N), (0, 0)))

  c = _sc_counts(dsts)
  p1 = _sc_aggregate(embp, srcs, dsts)
  h = _tc_layer(p1, c, embp, W1l.T, W1r.T, b1l.reshape(1, D), relu=True)
  p2 = _sc_aggregate(h, srcs, dsts)
  out = _tc_layer(p2, c, h, W2l.T, W2r.T, b2l.reshape(1, D), relu=False)
  return out[:N]


# speedup vs baseline: 2.1900x; 1.0593x over previous
import functools

import jax
import jax.numpy as jnp
from jax import lax
from jax.experimental import pallas as pl
from jax.experimental.pallas import tpu as pltpu
from jax.experimental.pallas import tpu_sc as plsc

N = 10000
E = 640000
D = 128

NC = 2
NS = 16
NW = NC * NS

CH = 128
GCH = 16
TOTCH = 5120
EPAD = TOTCH * CH
NP = 10240
HALF = NP // 2
SB = 4480
TROWS = 16
AROWS = HALF + TROWS
CPS = TOTCH // NS
NGRP = CPS // GCH
EBUF = GCH * CH + CH + 16
CPT = TOTCH // NW
BN = 1280


def _sc_counts(dsts):
  mesh = plsc.VectorSubcoreMesh(core_axis_name="c", subcore_axis_name="s")

  def body(dsts_h, c_h, dst_v, cnt_v):
    cid = lax.axis_index("c")
    sid = lax.axis_index("s")
    wid = sid * NC + cid
    start_chunk = wid * CPT

    zeros16 = jnp.zeros((16,), jnp.float32)
    @pl.loop(0, NP // 16)
    def _(i):
      cnt_v[pl.ds(i * 16, 16)] = zeros16

    ones16 = jnp.ones((16,), jnp.float32)
    @pl.loop(0, CPT // GCH)
    def _(g):
      pltpu.sync_copy(dsts_h.at[pl.ds(start_chunk + g * GCH, GCH)], dst_v)
      for j in range(GCH):
        for k in range(CH // 16):
          idx = dst_v[j, pl.ds(k * 16, 16)]
          plsc.addupdate_scatter(cnt_v, [idx], ones16)

    pltpu.sync_copy(cnt_v, c_h.at[wid])

  k = pl.kernel(body,
                out_type=jax.ShapeDtypeStruct((NW, NP), jnp.float32),
                mesh=mesh,
                scratch_types=(
                    pltpu.VMEM((GCH, CH), jnp.int32),
                    pltpu.VMEM((NP,), jnp.float32),
                ),
                compiler_params=pltpu.CompilerParams(
                    needs_layout_passes=False))
  return k(dsts)


def _sc_aggregate(table, srcs, dsts):
  mesh = plsc.VectorSubcoreMesh(core_axis_name="c", subcore_axis_name="s")

  def body(table_h, srcs_h, dsts_h, p_h, src_v, dst_v, rows_v, srow_v,
           esrc_v, edst_v, table_sh, acc_sh, semg, sems):
    cid = lax.axis_index("c")
    sid = lax.axis_index("s")
    c_lo = cid * SB
    c_len = jnp.where(cid == 0, SB, NP - SB)

    @pl.when(cid == 0)
    def _():
      rp = SB // NS
      pltpu.sync_copy(table_h.at[pl.ds(sid * rp, rp)],
                      table_sh.at[pl.ds(sid * rp, rp)])
    @pl.when(cid == 1)
    def _():
      rp = (NP - SB) // NS
      pltpu.sync_copy(table_h.at[pl.ds(SB + sid * rp, rp)],
                      table_sh.at[pl.ds(sid * rp, rp)])

    def drain(sem):
      pltpu.make_async_copy(table_h.at[pl.ds(0, CH)], rows_v.at[0],
                            sem).wait()

    trash16 = HALF + jnp.arange(16, dtype=jnp.int32)
    zeros16 = jnp.zeros((16,), jnp.float32)
    zeros16i = jnp.zeros((16,), jnp.int32)

    for q in range(2):
      q_lo = q * HALF

      @pl.loop(0, CH)
      def _(i):
        for k in range(D // 16):
          rows_v[0, i, pl.ds(k * 16, 16)] = zeros16
      for t in range(3):
        zc = sid + NS * t
        @pl.when(zc < HALF // CH)
        def _():
          pltpu.sync_copy(rows_v.at[0], acc_sh.at[pl.ds(zc * CH, CH)])
      @pl.when(sid == 0)
      def _():
        pltpu.sync_copy(rows_v.at[0].at[pl.ds(0, TROWS)],
                        acc_sh.at[pl.ds(HALF, TROWS)])

      plsc.subcore_barrier()

      @pl.loop(0, NGRP, init_carry=jnp.int32(0))
      def rem_out(g, cur):
        c0 = sid * CPS + g * GCH
        pltpu.sync_copy(srcs_h.at[pl.ds(c0, GCH)], src_v)
        pltpu.sync_copy(dsts_h.at[pl.ds(c0, GCH)], dst_v)

        for j in range(GCH):
          for k in range(CH // 16):
            sv = src_v[j, pl.ds(k * 16, 16)]
            dv = dst_v[j, pl.ds(k * 16, 16)]
            sl = sv - c_lo
            dl = dv - q_lo
            m = (plsc.bitcast(sl, jnp.uint32) < c_len.astype(jnp.uint32)) & (
                plsc.bitcast(dl, jnp.uint32) < jnp.uint32(HALF))
            plsc.store_compressed(esrc_v.at[pl.ds(cur, 16)], sl, mask=m)
            plsc.store_compressed(edst_v.at[pl.ds(cur, 16)], dl, mask=m)
            cur = cur + jnp.sum(m.astype(jnp.int32))

        nb = cur // CH

        @pl.when(nb > 0)
        def _():
          pltpu.async_copy(table_sh.at[esrc_v.at[pl.ds(0, CH)]],
                           rows_v.at[0], semg)

        @pl.loop(0, nb)
        def _(b):
          bb = lax.rem(b, 2)
          drain(semg)
          for k in range(CH // 16):
            srow_v[bb, pl.ds(k * 16, 16)] = edst_v[pl.ds(b * CH + k * 16,
                                                         16)]
          pltpu.async_copy(rows_v.at[bb], acc_sh.at[srow_v.at[bb]], sems,
                           add=True)
          @pl.when(b + 1 < nb)
          def _():
            @pl.when(b >= 1)
            def _():
              drain(sems)
            pltpu.async_copy(table_sh.at[esrc_v.at[pl.ds((b + 1) * CH,
                                                         CH)]],
                             rows_v.at[lax.rem(b + 1, 2)], semg)

        @pl.when(nb >= 2)
        def _():
          drain(sems)
        @pl.when(nb >= 1)
        def _():
          drain(sems)

        rem = cur - nb * CH
        @pl.when(nb > 0)
        def _():
          @pl.loop(0, (rem + 15) // 16)
          def _(i):
            esrc_v[pl.ds(i * 16, 16)] = esrc_v[pl.ds(nb * CH + i * 16, 16)]
            edst_v[pl.ds(i * 16, 16)] = edst_v[pl.ds(nb * CH + i * 16, 16)]
        return rem

      @pl.when(rem_out > 0)
      def _():
        @pl.loop(rem_out, CH, step=16)
        def _(i):
          esrc_v[pl.ds(i, 16)] = zeros16i
          edst_v[pl.ds(i, 16)] = trash16
        pltpu.sync_copy(table_sh.at[esrc_v.at[pl.ds(0, CH)]], rows_v.at[0])
        for k in range(CH // 16):
          srow_v[0, pl.ds(k * 16, 16)] = edst_v[pl.ds(k * 16, 16)]
        pltpu.sync_copy(rows_v.at[0], acc_sh.at[srow_v.at[0]], add=True)

      plsc.subcore_barrier()

      rps = HALF // NS
      pltpu.sync_copy(acc_sh.at[pl.ds(sid * rps, rps)],
                      p_h.at[cid, pl.ds(q_lo + sid * rps, rps)])
      plsc.subcore_barrier()

  k = pl.kernel(body,
                out_type=jax.ShapeDtypeStruct((NC, NP, D), jnp.float32),
                mesh=mesh,
                scratch_types=(
                    pltpu.VMEM((GCH, CH), jnp.int32),
                    pltpu.VMEM((GCH, CH), jnp.int32),
                    pltpu.VMEM((2, CH, D), jnp.float32),
                    pltpu.VMEM((2, CH), jnp.int32),
                    pltpu.VMEM((EBUF,), jnp.int32),
                    pltpu.VMEM((EBUF,), jnp.int32),
                    pltpu.VMEM_SHARED((NP - SB, D), jnp.float32),
                    pltpu.VMEM_SHARED((AROWS, D), jnp.float32),
                    pltpu.SemaphoreType.DMA,
                    pltpu.SemaphoreType.DMA,
                ),
                compiler_params=pltpu.CompilerParams(
                    needs_layout_passes=False))
  return k(table, srcs, dsts)


def _tc_layer_body(relu, p_ref, c_ref, x_ref, wl_ref, wr_ref, b_ref, o_ref):
  s = p_ref[0] + p_ref[1]
  cnt = jnp.sum(c_ref[...], axis=0)[:, None]
  mean = s / jnp.maximum(cnt, 1.0)
  acc = (jnp.dot(mean, wl_ref[...], preferred_element_type=jnp.float32)
         + jnp.dot(x_ref[...], wr_ref[...], preferred_element_type=jnp.float32)
         + b_ref[...])
  o_ref[...] = jnp.maximum(acc, 0.0) if relu else acc


def _tc_layer(p, c, x, wlt, wrt, b, relu):
  return pl.pallas_call(
      functools.partial(_tc_layer_body, relu),
      grid=(NP // BN,),
      in_specs=[
          pl.BlockSpec((NC, BN, D), lambda i: (0, i, 0)),
          pl.BlockSpec((NW, BN), lambda i: (0, i)),
          pl.BlockSpec((BN, D), lambda i: (i, 0)),
          pl.BlockSpec((D, D), lambda i: (0, 0)),
          pl.BlockSpec((D, D), lambda i: (0, 0)),
          pl.BlockSpec((1, D), lambda i: (0, 0)),
      ],
      out_specs=pl.BlockSpec((BN, D), lambda i: (i, 0)),
      out_shape=jax.ShapeDtypeStruct((NP, D), jnp.float32),
  )(p, c, x, wlt, wrt, b)


def kernel(edge_index, emb, W1l, b1l, W1r, W2l, b2l, W2r):
  src = edge_index[0]
  dst = edge_index[1]
  pad = EPAD - E
  srcs = jnp.concatenate(
      [src, jnp.zeros((pad,), jnp.int32)]).reshape(TOTCH, CH)
  pad_dst = N + jnp.arange(pad, dtype=jnp.int32) % (NP - N)
  dsts = jnp.concatenate([dst, pad_dst]).reshape(TOTCH, CH)
  embp = jnp.pad(emb, ((0, NP - N), (0, 0)))

  c = _sc_counts(dsts)
  p1 = _sc_aggregate(embp, srcs, dsts)
  h = _tc_layer(p1, c, embp, W1l.T, W1r.T, b1l.reshape(1, D), relu=True)
  p2 = _sc_aggregate(h, srcs, dsts)
  out = _tc_layer(p2, c, h, W2l.T, W2r.T, b2l.reshape(1, D), relu=False)
  return out[:N]
